# baseline (speedup 1.0000x reference)
import functools

import jax
import jax.numpy as jnp
from jax import lax
from jax.experimental import pallas as pl
from jax.experimental.pallas import tpu as pltpu


def _fused_kernel(x_ref, g_ref, be_ref, w_ref, b_ref,
                  tok_ref, emb_ref, wle_ref, c_ref, ble_ref,
                  xo_ref, mo_ref, *, cs, csn, vocab):
    for e in range(x_ref.shape[0]):
        _one_element(x_ref, g_ref, be_ref, w_ref, b_ref, tok_ref, emb_ref,
                     wle_ref, c_ref, ble_ref, xo_ref, mo_ref, e,
                     cs=cs, csn=csn, vocab=vocab)


def _one_element(x_ref, g_ref, be_ref, w_ref, b_ref,
                 tok_ref, emb_ref, wle_ref, c_ref, ble_ref,
                 xo_ref, mo_ref, e, *, cs, csn, vocab):
    _, Lr, cin, Lc = x_ref.shape
    rows = Lr * cin
    x = x_ref[e].reshape(rows, Lc)

    rmod = lax.broadcasted_iota(jnp.int32, (cin, rows), 1) % cin
    cidx = lax.broadcasted_iota(jnp.int32, (cin, rows), 0)
    sel = (rmod == cidx).astype(jnp.float32)
    s1 = jnp.dot(sel, x, preferred_element_type=jnp.float32)
    s2 = jnp.dot(sel, x * x, preferred_element_type=jnp.float32)
    inv_n = 1.0 / (Lr * Lc)
    mean = jnp.sum(s1, axis=1, keepdims=True) * inv_n
    ex2 = jnp.sum(s2, axis=1, keepdims=True) * inv_n
    var = jnp.maximum(ex2 - mean * mean, 0.0)
    rstd = lax.rsqrt(var + 1e-5)
    scale = g_ref[...] * rstd
    shift = be_ref[...] - mean * scale

    rmod_t = lax.broadcasted_iota(jnp.int32, (rows, cin), 0) % cin
    cidx_t = lax.broadcasted_iota(jnp.int32, (rows, cin), 1)
    sel_t = (rmod_t == cidx_t).astype(jnp.float32)
    scale_col = jnp.dot(sel_t, scale, preferred_element_type=jnp.float32)
    shift_col = jnp.dot(sel_t, shift, preferred_element_type=jnp.float32)

    xa = x * scale_col + shift_col
    xe = jnp.where(xa > 0, xa, jnp.exp(jnp.minimum(xa, 0.0)) - 1.0)

    d = xo_ref.shape[2]
    crows = cs * cin
    for i in range(Lr // cs):
        y = jnp.dot(w_ref[...], xe[i * crows:(i + 1) * crows, :],
                    preferred_element_type=jnp.float32) + b_ref[...]
        xo_ref[e, i * cs:(i + 1) * cs] = y.reshape(cs, d, Lc)

    _, N, demb, Lm = emb_ref.shape
    emb = emb_ref[e].reshape(N * demb, Lm)
    tok = tok_ref[e]
    for i in range(N // csn):
        masks = jnp.concatenate(
            [(tok[i * csn:(i + 1) * csn] == t).astype(jnp.float32)
             for t in range(vocab)], axis=0)
        y = (jnp.dot(wle_ref[...], emb[i * csn * demb:(i + 1) * csn * demb, :],
                     preferred_element_type=jnp.float32)
             + jnp.dot(c_ref[...], masks, preferred_element_type=jnp.float32)
             + ble_ref[...])
        mo_ref[e, i * csn:(i + 1) * csn] = y.reshape(csn, d, Lm)


def kernel(f2d, msa, msa_emb, gamma, beta, w_conv, b_conv, table, w_le, b_le):
    f32 = jnp.float32
    B, L, _, cin = f2d.shape
    d = w_conv.shape[0]
    Bm, Nm, Lm = msa.shape
    demb = msa_emb.shape[-1]
    vocab = table.shape[0]

    x_t = jnp.transpose(f2d.astype(f32), (0, 1, 3, 2))
    emb_t = jnp.transpose(msa_emb.astype(f32), (0, 1, 3, 2))

    cs = 4 if L % 4 == 0 else L
    w_blk = jnp.kron(jnp.eye(cs, dtype=f32), w_conv.astype(f32))
    b_col = jnp.tile(b_conv.astype(f32), cs).reshape(cs * d, 1)

    csn = 4 if Nm % 4 == 0 else Nm
    wle_blk = jnp.kron(jnp.eye(csn, dtype=f32), w_le.astype(f32))
    ble_col = jnp.tile(b_le.astype(f32), csn).reshape(csn * d, 1)
    eye_cs = jnp.eye(csn, dtype=f32)
    c_cat = jnp.concatenate(
        [jnp.kron(eye_cs, table[t].astype(f32)[:, None]) for t in range(vocab)],
        axis=1)

    eb = 2 if B % 2 == 0 else 1
    x_out, m_out = pl.pallas_call(
        functools.partial(_fused_kernel, cs=cs, csn=csn, vocab=vocab),
        out_shape=(jax.ShapeDtypeStruct((B, L, d, L), f32),
                   jax.ShapeDtypeStruct((Bm, Nm, d, Lm), f32)),
        grid=(B // eb,),
        in_specs=[
            pl.BlockSpec((eb, L, cin, L), lambda b: (b, 0, 0, 0)),
            pl.BlockSpec((cin, 1), lambda b: (0, 0)),
            pl.BlockSpec((cin, 1), lambda b: (0, 0)),
            pl.BlockSpec((cs * d, cs * cin), lambda b: (0, 0)),
            pl.BlockSpec((cs * d, 1), lambda b: (0, 0)),
            pl.BlockSpec((eb, Nm, Lm), lambda b: (b, 0, 0)),
            pl.BlockSpec((eb, Nm, demb, Lm), lambda b: (b, 0, 0, 0)),
            pl.BlockSpec((csn * d, csn * demb), lambda b: (0, 0)),
            pl.BlockSpec((csn * d, vocab * csn), lambda b: (0, 0)),
            pl.BlockSpec((csn * d, 1), lambda b: (0, 0)),
        ],
        out_specs=(pl.BlockSpec((eb, L, d, L), lambda b: (b, 0, 0, 0)),
                   pl.BlockSpec((eb, Nm, d, Lm), lambda b: (b, 0, 0, 0))),
        compiler_params=pltpu.CompilerParams(
            dimension_semantics=("parallel",),
            vmem_limit_bytes=100 * 1024 * 1024,
        ),
    )(x_t, gamma.astype(f32).reshape(cin, 1), beta.astype(f32).reshape(cin, 1),
      w_blk, b_col, msa.astype(jnp.int32), emb_t, wle_blk, c_cat, ble_col)

    x = jnp.transpose(x_out, (0, 1, 3, 2))
    m = jnp.transpose(m_out, (0, 1, 3, 2))
    return x, m
